# Initial kernel scaffold; baseline (speedup 1.0000x reference)
import jax
import jax.numpy as jnp
from jax.experimental import pallas as pl


def kernel(x, edge_index, edge_vals, W):
    raise NotImplementedError("write your pallas kernel here")


# speedup vs baseline: 4.3951x; 4.3951x over previous
import functools

import jax
import jax.numpy as jnp
from jax import lax
from jax.experimental import pallas as pl
from jax.experimental.pallas import tpu as pltpu
from jax.experimental.pallas import tpu_sc as plsc

N_NODES = 10000
N_PAD = 10240
D = 128
N_EDGES = 320000
NC = 2
NS = 16
CH = 80
EDGES_PER_TILE = N_EDGES // (NC * NS)
NCHUNK = EDGES_PER_TILE // CH
ROWS_PER_TILE = N_PAD // NS
ZROWS = 128
LANES = 16


def _mm_body(x_ref, w_ref, o_ref):
    o_ref[...] = jnp.dot(x_ref[...], w_ref[...],
                         preferred_element_type=jnp.float32)


def _combine_body(p_ref, o_ref):
    o_ref[...] = jnp.maximum(p_ref[0] + p_ref[1], 0.0)


def _bcast_lane(vec, lane):
    idx = jnp.full((LANES, 1), lane, jnp.int32)
    dnums = lax.GatherDimensionNumbers(
        offset_dims=(), collapsed_slice_dims=(0,), start_index_map=(0,))
    return lax.gather(vec, idx, dnums, (1,),
                      mode=lax.GatherScatterMode.PROMISE_IN_BOUNDS)


def _sc_scatter_body(xw, srcs, dsts, evals, out,
                     src_v, dst_v, val_v, rows_v, zbuf, acc, sem):
    c = lax.axis_index("c")
    s = lax.axis_index("s")
    tile_edge_base = (c * NS + s) * EDGES_PER_TILE

    def zrow(r, carry):
        for j in range(D // LANES):
            zbuf[r, pl.ds(j * LANES, LANES)] = jnp.zeros((LANES,), jnp.float32)
        return carry
    lax.fori_loop(0, ZROWS, zrow, 0)
    for k in range(ROWS_PER_TILE // ZROWS):
        pltpu.sync_copy(
            zbuf, acc.at[pl.ds(s * ROWS_PER_TILE + k * ZROWS, ZROWS)])
    plsc.subcore_barrier()

    def chunk_body(ch, carry):
        base = tile_edge_base + ch * CH
        pltpu.sync_copy(srcs.at[pl.ds(base, CH)], src_v)
        pltpu.sync_copy(dsts.at[pl.ds(base, CH)], dst_v)
        pltpu.sync_copy(evals.at[pl.ds(base, CH)], val_v)
        pltpu.async_copy(xw.at[src_v], rows_v, sem).wait()

        def group_body(g, gcarry):
            vals16 = val_v[pl.ds(g * LANES, LANES)]
            for l in range(LANES):
                vv = _bcast_lane(vals16, l)
                e = g * LANES + l
                for j in range(D // LANES):
                    sl = pl.ds(j * LANES, LANES)
                    rows_v[e, sl] = rows_v[e, sl] * vv
            return gcarry
        lax.fori_loop(0, CH // LANES, group_body, 0)

        pltpu.sync_copy(rows_v, acc.at[dst_v], add=True)
        return carry
    lax.fori_loop(0, NCHUNK, chunk_body, 0)

    plsc.subcore_barrier()
    pltpu.sync_copy(acc.at[pl.ds(s * ROWS_PER_TILE, ROWS_PER_TILE)],
                    out.at[c, pl.ds(s * ROWS_PER_TILE, ROWS_PER_TILE)])


_sc_scatter = functools.partial(
    pl.kernel,
    mesh=plsc.VectorSubcoreMesh(core_axis_name="c", subcore_axis_name="s"),
    out_type=jax.ShapeDtypeStruct((NC, N_PAD, D), jnp.float32),
    scratch_types=[
        pltpu.VMEM((CH,), jnp.int32),
        pltpu.VMEM((CH,), jnp.int32),
        pltpu.VMEM((CH,), jnp.float32),
        pltpu.VMEM((CH, D), jnp.float32),
        pltpu.VMEM((ZROWS, D), jnp.float32),
        pltpu.VMEM_SHARED((N_PAD, D), jnp.float32),
        pltpu.SemaphoreType.DMA,
    ],
)(_sc_scatter_body)


def kernel(x, edge_index, edge_vals, W):
    xw = pl.pallas_call(
        _mm_body,
        grid=(10,),
        in_specs=[
            pl.BlockSpec((N_NODES // 10, D), lambda i: (i, 0)),
            pl.BlockSpec((D, D), lambda i: (0, 0)),
        ],
        out_specs=pl.BlockSpec((N_NODES // 10, D), lambda i: (i, 0)),
        out_shape=jax.ShapeDtypeStruct((N_NODES, D), jnp.float32),
    )(x, W)

    src = edge_index[1].astype(jnp.int32)
    dst = edge_index[0].astype(jnp.int32)

    partials = _sc_scatter(xw, src, dst, edge_vals)

    out = pl.pallas_call(
        _combine_body,
        grid=(10,),
        in_specs=[pl.BlockSpec((NC, N_NODES // 10, D), lambda i: (0, i, 0))],
        out_specs=pl.BlockSpec((N_NODES // 10, D), lambda i: (i, 0)),
        out_shape=jax.ShapeDtypeStruct((N_NODES, D), jnp.float32),
    )(partials[:, :N_NODES])
    return out


# speedup vs baseline: 7.4279x; 1.6900x over previous
import functools

import jax
import jax.numpy as jnp
from jax import lax
from jax.experimental import pallas as pl
from jax.experimental.pallas import tpu as pltpu
from jax.experimental.pallas import tpu_sc as plsc

N_NODES = 10000
N_PAD = 10240
D = 128
N_EDGES = 320000
NC = 2
NS = 16
NW = NC * NS
CH = 64
NCHUNK = N_EDGES // CH
ROWS_PER_TILE = N_PAD // NS
LANES = 16
T_MAX = 158


def _mm_body(x_ref, w_ref, o_ref):
    o_ref[...] = jnp.dot(x_ref[...], w_ref[...],
                         preferred_element_type=jnp.float32)


def _combine_body(p_ref, o_ref):
    o_ref[...] = jnp.maximum(p_ref[0] + p_ref[1], 0.0)


def _bcast_lane(vec, lane):
    idx = jnp.full((LANES, 1), lane, jnp.int32)
    dnums = lax.GatherDimensionNumbers(
        offset_dims=(), collapsed_slice_dims=(0,), start_index_map=(0,))
    return lax.gather(vec, idx, dnums, (1,),
                      mode=lax.GatherScatterMode.PROMISE_IN_BOUNDS)


def _sc_scatter_body(xw, packed, out,
                     ibuf0, ibuf1, rows0, rows1, acc,
                     isem0, isem1, gsem0, gsem1):
    c = lax.axis_index("c")
    s = lax.axis_index("s")
    w = c * NS + s
    n_w = 156 + jnp.where(w < NCHUNK - 156 * NW, 1, 0)
    ibuf = (ibuf0, ibuf1)
    rows = (rows0, rows1)
    isem = (isem0, isem1)
    gsem = (gsem0, gsem1)

    def zrow(r, carry):
        for j in range(D // LANES):
            rows0[r, pl.ds(j * LANES, LANES)] = jnp.zeros((LANES,),
                                                          jnp.float32)
        return carry
    lax.fori_loop(0, CH, zrow, 0)
    for k in range(ROWS_PER_TILE // CH):
        pltpu.sync_copy(rows0, acc.at[pl.ds(s * ROWS_PER_TILE + k * CH, CH)])

    pltpu.sync_copy(packed.at[w], ibuf0)
    pltpu.async_copy(xw.at[ibuf0.at[0]], rows0, gsem0)
    pltpu.async_copy(packed.at[w + NW], ibuf1, isem1)

    plsc.subcore_barrier()

    def outer_body(i, carry):
        for p in range(2):
            t = i * 2 + p
            q = 1 - p

            @pl.when(t + 1 < n_w)
            def _():
                pltpu.make_async_copy(packed.at[w], ibuf[q], isem[q]).wait()
                pltpu.async_copy(xw.at[ibuf[q].at[0]], rows[q], gsem[q])

            @pl.when(t < n_w)
            def _():
                pltpu.make_async_copy(xw.at[ibuf[p].at[0]], rows[p],
                                      gsem[p]).wait()

                def group_body(g, gcarry):
                    vbits = ibuf[p][2, pl.ds(g * LANES, LANES)]
                    vals16 = lax.bitcast_convert_type(vbits, jnp.float32)
                    for l in range(LANES):
                        vv = _bcast_lane(vals16, l)
                        e = g * LANES + l
                        for j in range(D // LANES):
                            sl = pl.ds(j * LANES, LANES)
                            rows[p][e, sl] = rows[p][e, sl] * vv
                    return gcarry
                lax.fori_loop(0, CH // LANES, group_body, 0)

                pltpu.sync_copy(rows[p], acc.at[ibuf[p].at[1]], add=True)

            @pl.when(t + 2 < n_w)
            def _():
                pltpu.async_copy(packed.at[w + (t + 2) * NW], ibuf[p],
                                 isem[p])
        return carry
    lax.fori_loop(0, T_MAX // 2, outer_body, 0)

    plsc.subcore_barrier()
    pltpu.sync_copy(acc.at[pl.ds(s * ROWS_PER_TILE, ROWS_PER_TILE)],
                    out.at[c, pl.ds(s * ROWS_PER_TILE, ROWS_PER_TILE)])


_sc_scatter = functools.partial(
    pl.kernel,
    mesh=plsc.VectorSubcoreMesh(core_axis_name="c", subcore_axis_name="s"),
    out_type=jax.ShapeDtypeStruct((NC, N_PAD, D), jnp.float32),
    scratch_types=[
        pltpu.VMEM((3, CH), jnp.int32),
        pltpu.VMEM((3, CH), jnp.int32),
        pltpu.VMEM((CH, D), jnp.float32),
        pltpu.VMEM((CH, D), jnp.float32),
        pltpu.VMEM_SHARED((N_PAD, D), jnp.float32),
        pltpu.SemaphoreType.DMA,
        pltpu.SemaphoreType.DMA,
        pltpu.SemaphoreType.DMA,
        pltpu.SemaphoreType.DMA,
    ],
)(_sc_scatter_body)


def kernel(x, edge_index, edge_vals, W):
    xw = pl.pallas_call(
        _mm_body,
        grid=(10,),
        in_specs=[
            pl.BlockSpec((N_NODES // 10, D), lambda i: (i, 0)),
            pl.BlockSpec((D, D), lambda i: (0, 0)),
        ],
        out_specs=pl.BlockSpec((N_NODES // 10, D), lambda i: (i, 0)),
        out_shape=jax.ShapeDtypeStruct((N_NODES, D), jnp.float32),
    )(x, W)

    src = edge_index[1].astype(jnp.int32).reshape(NCHUNK, CH)
    dst = edge_index[0].astype(jnp.int32).reshape(NCHUNK, CH)
    vbits = lax.bitcast_convert_type(edge_vals, jnp.int32).reshape(NCHUNK, CH)
    packed = jnp.stack([src, dst, vbits], axis=1)

    partials = _sc_scatter(xw, packed)

    out = pl.pallas_call(
        _combine_body,
        grid=(10,),
        in_specs=[pl.BlockSpec((NC, N_NODES // 10, D), lambda i: (0, i, 0))],
        out_specs=pl.BlockSpec((N_NODES // 10, D), lambda i: (i, 0)),
        out_shape=jax.ShapeDtypeStruct((N_NODES, D), jnp.float32),
    )(partials[:, :N_NODES])
    return out


# speedup vs baseline: 10.5701x; 1.4230x over previous
import functools

import jax
import jax.numpy as jnp
from jax import lax
from jax.experimental import pallas as pl
from jax.experimental.pallas import tpu as pltpu
from jax.experimental.pallas import tpu_sc as plsc

N_NODES = 10000
N_PAD = 10240
D = 128
N_EDGES = 320000
NC = 2
NS = 16
NW = NC * NS
CH = 64
NCHUNK = N_EDGES // CH
ROWS_PER_TILE = N_PAD // NS
LANES = 16
NB = 3
T_MAX = 159


def _mm_body(x_ref, w_ref, o_ref):
    o_ref[...] = jnp.dot(x_ref[...], w_ref[...],
                         preferred_element_type=jnp.float32)


def _combine_body(p_ref, o_ref):
    o_ref[...] = jnp.maximum(p_ref[0] + p_ref[1], 0.0)


def _bcast_lane(vec, lane):
    idx = jnp.full((LANES, 1), lane, jnp.int32)
    dnums = lax.GatherDimensionNumbers(
        offset_dims=(), collapsed_slice_dims=(0,), start_index_map=(0,))
    return lax.gather(vec, idx, dnums, (1,),
                      mode=lax.GatherScatterMode.PROMISE_IN_BOUNDS)


def _sc_scatter_body(xw, packed, out,
                     ibuf0, ibuf1, ibuf2, rows0, rows1, rows2,
                     dbuf0, dbuf1, dbuf2, acc,
                     isem0, isem1, isem2, gsem0, gsem1, gsem2,
                     ssem0, ssem1, ssem2):
    c = lax.axis_index("c")
    s = lax.axis_index("s")
    w = c * NS + s
    n_w = 156 + jnp.where(w < NCHUNK - 156 * NW, 1, 0)
    ibuf = (ibuf0, ibuf1, ibuf2)
    rows = (rows0, rows1, rows2)
    dbuf = (dbuf0, dbuf1, dbuf2)
    isem = (isem0, isem1, isem2)
    gsem = (gsem0, gsem1, gsem2)
    ssem = (ssem0, ssem1, ssem2)

    def zrow(r, carry):
        for j in range(D // LANES):
            rows0[r, pl.ds(j * LANES, LANES)] = jnp.zeros((LANES,),
                                                          jnp.float32)
        return carry
    lax.fori_loop(0, CH, zrow, 0)
    for k in range(ROWS_PER_TILE // CH):
        pltpu.sync_copy(rows0, acc.at[pl.ds(s * ROWS_PER_TILE + k * CH, CH)])

    for k in range(NB):
        pltpu.async_copy(packed.at[w + k * NW], ibuf[k], isem[k])
    pltpu.make_async_copy(packed.at[w], ibuf0, isem0).wait()
    pltpu.async_copy(xw.at[ibuf0.at[0]], rows0, gsem0)

    plsc.subcore_barrier()

    def outer_body(i, carry):
        for p in range(NB):
            t = i * NB + p
            q = (p + 1) % NB

            @pl.when(t + 1 < n_w)
            def _():
                pltpu.make_async_copy(packed.at[w], ibuf[q], isem[q]).wait()

                @pl.when(t >= 2)
                def _():
                    pltpu.make_async_copy(
                        rows[q], acc.at[dbuf[q].at[0]], ssem[q]).wait()

                pltpu.async_copy(xw.at[ibuf[q].at[0]], rows[q], gsem[q])

            @pl.when(t < n_w)
            def _():
                pltpu.make_async_copy(xw.at[ibuf[p].at[0]], rows[p],
                                      gsem[p]).wait()

                def group_body(g, gcarry):
                    vbits = ibuf[p][2, pl.ds(g * LANES, LANES)]
                    vals16 = lax.bitcast_convert_type(vbits, jnp.float32)
                    for l in range(LANES):
                        vv = _bcast_lane(vals16, l)
                        e = g * LANES + l
                        for j in range(D // LANES):
                            sl = pl.ds(j * LANES, LANES)
                            rows[p][e, sl] = rows[p][e, sl] * vv
                    return gcarry
                lax.fori_loop(0, CH // LANES, group_body, 0)

                for j in range(CH // LANES):
                    sl = pl.ds(j * LANES, LANES)
                    dbuf[p][0, sl] = ibuf[p][1, sl]
                pltpu.async_copy(rows[p], acc.at[dbuf[p].at[0]], ssem[p],
                                 add=True)

            @pl.when(t + NB < n_w)
            def _():
                pltpu.async_copy(packed.at[w + (t + NB) * NW], ibuf[p],
                                 isem[p])
        return carry
    lax.fori_loop(0, T_MAX // NB, outer_body, 0)

    for p in range(NB):
        pltpu.make_async_copy(rows[p], acc.at[dbuf[p].at[0]],
                              ssem[p]).wait()
    plsc.subcore_barrier()
    pltpu.sync_copy(acc.at[pl.ds(s * ROWS_PER_TILE, ROWS_PER_TILE)],
                    out.at[c, pl.ds(s * ROWS_PER_TILE, ROWS_PER_TILE)])


_sc_scatter = functools.partial(
    pl.kernel,
    mesh=plsc.VectorSubcoreMesh(core_axis_name="c", subcore_axis_name="s"),
    out_type=jax.ShapeDtypeStruct((NC, N_PAD, D), jnp.float32),
    scratch_types=(
        [pltpu.VMEM((3, CH), jnp.int32) for _ in range(NB)]
        + [pltpu.VMEM((CH, D), jnp.float32) for _ in range(NB)]
        + [pltpu.VMEM((1, CH), jnp.int32) for _ in range(NB)]
        + [pltpu.VMEM_SHARED((N_PAD, D), jnp.float32)]
        + [pltpu.SemaphoreType.DMA for _ in range(3 * NB)]
    ),
)(_sc_scatter_body)


def kernel(x, edge_index, edge_vals, W):
    xw = pl.pallas_call(
        _mm_body,
        grid=(10,),
        in_specs=[
            pl.BlockSpec((N_NODES // 10, D), lambda i: (i, 0)),
            pl.BlockSpec((D, D), lambda i: (0, 0)),
        ],
        out_specs=pl.BlockSpec((N_NODES // 10, D), lambda i: (i, 0)),
        out_shape=jax.ShapeDtypeStruct((N_NODES, D), jnp.float32),
    )(x, W)

    src = edge_index[1].astype(jnp.int32).reshape(NCHUNK, CH)
    dst = edge_index[0].astype(jnp.int32).reshape(NCHUNK, CH)
    vbits = lax.bitcast_convert_type(edge_vals, jnp.int32).reshape(NCHUNK, CH)
    packed = jnp.stack([src, dst, vbits], axis=1)

    partials = _sc_scatter(xw, packed)

    out = pl.pallas_call(
        _combine_body,
        grid=(10,),
        in_specs=[pl.BlockSpec((NC, N_NODES // 10, D), lambda i: (0, i, 0))],
        out_specs=pl.BlockSpec((N_NODES // 10, D), lambda i: (i, 0)),
        out_shape=jax.ShapeDtypeStruct((N_NODES, D), jnp.float32),
    )(partials)
    return out
